# baseline (speedup 1.0000x reference)
import functools

import jax
import jax.numpy as jnp
from jax import lax
from jax.experimental import pallas as pl
from jax.experimental.pallas import tpu as pltpu
from jax.experimental.pallas import tpu_sc as plsc

D_EMBED = 64
D_MODEL = 128

NUM_CORES = 2
NUM_SUBCORES = 16
NUM_WORKERS = NUM_CORES * NUM_SUBCORES

CHUNK = 1024


def _gather_kernel(n_tokens: int):
    per_w = n_tokens // NUM_WORKERS
    steps = per_w // CHUNK
    mesh = plsc.VectorSubcoreMesh(core_axis_name="c", subcore_axis_name="s")

    @functools.partial(
        pl.kernel,
        mesh=mesh,
        out_type=jax.ShapeDtypeStruct((n_tokens, D_EMBED), jnp.float32),
        scratch_types=[
            pltpu.VMEM((CHUNK,), jnp.int32),
            pltpu.VMEM((CHUNK, D_EMBED), jnp.float32),
            pltpu.SemaphoreType.DMA,
        ],
        compiler_params=pltpu.CompilerParams(use_tc_tiling_on_sc=False),
    )
    def body(idx_hbm, tab_hbm, out_hbm, idx_v, rows_v, sem):
        wid = lax.axis_index("s") * NUM_CORES + lax.axis_index("c")
        base = wid * per_w

        def step(i, carry):
            off = base + i * CHUNK
            pltpu.sync_copy(idx_hbm.at[pl.ds(off, CHUNK)], idx_v)
            pltpu.async_copy(tab_hbm.at[idx_v], rows_v, sem).wait()
            pltpu.sync_copy(rows_v, out_hbm.at[pl.ds(off, CHUNK)])
            return carry

        lax.fori_loop(0, steps, step, 0)

    return body


def _proj_block(h_ref, wt_ref, o_ref):
    o_ref[...] = jnp.dot(h_ref[...], wt_ref[...],
                         preferred_element_type=jnp.float32)


def _projection(h, wt, blk=2048):
    n = h.shape[0]
    grid = n // blk
    return pl.pallas_call(
        _proj_block,
        grid=(grid,),
        in_specs=[
            pl.BlockSpec((blk, D_EMBED), lambda i: (i, 0)),
            pl.BlockSpec((D_EMBED, D_MODEL), lambda i: (0, 0)),
        ],
        out_specs=pl.BlockSpec((blk, D_MODEL), lambda i: (i, 0)),
        out_shape=jax.ShapeDtypeStruct((n, D_MODEL), jnp.float32),
    )(h, wt)


def kernel(x, emb_table, W_proj):
    b, l = x.shape
    n = b * l
    xf = x.reshape(n).astype(jnp.int32)
    h = _gather_kernel(n)(xf, emb_table)
    out = _projection(h, W_proj.T)
    return out.reshape(b, l, D_MODEL)


# speedup vs baseline: 1.3337x; 1.3337x over previous
import functools

import jax
import jax.numpy as jnp
from jax import lax
from jax.experimental import pallas as pl
from jax.experimental.pallas import tpu as pltpu
from jax.experimental.pallas import tpu_sc as plsc

D_EMBED = 64
D_MODEL = 128

NUM_CORES = 2
NUM_SUBCORES = 16
NUM_WORKERS = NUM_CORES * NUM_SUBCORES

CHUNK = 1024


def _gather_kernel(n_tokens: int):
    per_w = n_tokens // NUM_WORKERS
    steps = per_w // CHUNK
    mesh = plsc.VectorSubcoreMesh(core_axis_name="c", subcore_axis_name="s")

    @functools.partial(
        pl.kernel,
        mesh=mesh,
        out_type=jax.ShapeDtypeStruct((n_tokens, D_EMBED), jnp.float32),
        scratch_types=[
            pltpu.VMEM((CHUNK,), jnp.int32),
            pltpu.VMEM((CHUNK, D_EMBED), jnp.float32),
            pltpu.SemaphoreType.DMA,
        ],
        compiler_params=pltpu.CompilerParams(use_tc_tiling_on_sc=False),
    )
    def body(idx_hbm, tab_hbm, out_hbm, idx_v, rows_v, sem):
        wid = lax.axis_index("s") * NUM_CORES + lax.axis_index("c")
        base = wid * per_w

        def step(i, carry):
            off = base + i * CHUNK
            pltpu.sync_copy(idx_hbm.at[pl.ds(off, CHUNK)], idx_v)
            pltpu.async_copy(tab_hbm.at[idx_v], rows_v, sem).wait()
            pltpu.sync_copy(rows_v, out_hbm.at[pl.ds(off, CHUNK)])
            return carry

        lax.fori_loop(0, steps, step, 0)

    return body


def _proj_block(h_ref, wt_ref, o_ref):
    o_ref[...] = jnp.dot(h_ref[...], wt_ref[...],
                         preferred_element_type=jnp.float32)


def _projection(h, wt, blk=2048):
    n = h.shape[0]
    grid = n // blk
    return pl.pallas_call(
        _proj_block,
        grid=(grid,),
        in_specs=[
            pl.BlockSpec((blk, D_EMBED), lambda i: (i, 0)),
            pl.BlockSpec((D_EMBED, D_MODEL), lambda i: (0, 0)),
        ],
        out_specs=pl.BlockSpec((blk, D_MODEL), lambda i: (i, 0)),
        out_shape=jax.ShapeDtypeStruct((n, D_MODEL), jnp.float32),
    )(h, wt)


def kernel(x, emb_table, W_proj):
    b, l = x.shape
    n = b * l
    xf = x.reshape(n).astype(jnp.int32)
    h = _gather_kernel(n)(xf, emb_table)
    return h.reshape(b, l, D_EMBED)
    out = _projection(h, W_proj.T)
    return out.reshape(b, l, D_MODEL)


# speedup vs baseline: 2.0227x; 1.5166x over previous
import functools

import jax
import jax.numpy as jnp
from jax import lax
from jax.experimental import pallas as pl
from jax.experimental.pallas import tpu as pltpu
from jax.experimental.pallas import tpu_sc as plsc

D_EMBED = 64
D_MODEL = 128

NUM_CORES = 2
NUM_SUBCORES = 16
NUM_WORKERS = NUM_CORES * NUM_SUBCORES

CHUNK = 512


def _gather_kernel(n_tokens: int):
    per_w = n_tokens // NUM_WORKERS
    steps = per_w // CHUNK
    mesh = plsc.VectorSubcoreMesh(core_axis_name="c", subcore_axis_name="s")

    @functools.partial(
        pl.kernel,
        mesh=mesh,
        out_type=jax.ShapeDtypeStruct((n_tokens, D_MODEL), jnp.float32),
        scratch_types=[
            pltpu.VMEM((CHUNK,), jnp.int32),
            pltpu.VMEM((CHUNK, D_MODEL), jnp.float32),
            pltpu.SemaphoreType.DMA,
        ],
    )
    def body(idx_hbm, tab_hbm, out_hbm, idx_v, rows_v, sem):
        wid = lax.axis_index("s") * NUM_CORES + lax.axis_index("c")
        base = wid * per_w

        def step(i, carry):
            off = base + i * CHUNK
            pltpu.sync_copy(idx_hbm.at[pl.ds(off, CHUNK)], idx_v)
            pltpu.async_copy(tab_hbm.at[idx_v], rows_v, sem).wait()
            pltpu.sync_copy(rows_v, out_hbm.at[pl.ds(off, CHUNK)])
            return carry

        lax.fori_loop(0, steps, step, 0)

    return body


def _proj_block(t_ref, wt_ref, o_ref):
    o_ref[...] = jnp.dot(t_ref[...], wt_ref[...],
                         preferred_element_type=jnp.float32)


def _project_table(tab, wt, blk=4096):
    v = tab.shape[0]
    grid = v // blk
    return pl.pallas_call(
        _proj_block,
        grid=(grid,),
        in_specs=[
            pl.BlockSpec((blk, D_EMBED), lambda i: (i, 0)),
            pl.BlockSpec((D_EMBED, D_MODEL), lambda i: (0, 0)),
        ],
        out_specs=pl.BlockSpec((blk, D_MODEL), lambda i: (i, 0)),
        out_shape=jax.ShapeDtypeStruct((v, D_MODEL), jnp.float32),
    )(tab, wt)


def kernel(x, emb_table, W_proj):
    b, l = x.shape
    n = b * l
    xf = x.reshape(n).astype(jnp.int32)
    tabp = _project_table(emb_table, W_proj.T)
    out = _gather_kernel(n)(xf, tabp)
    return out.reshape(b, l, D_MODEL)
